# Initial kernel scaffold; baseline (speedup 1.0000x reference)
import functools

import jax
import jax.numpy as jnp
from jax import lax
from jax.experimental import pallas as pl
from jax.experimental.pallas import tpu as pltpu
from jax.experimental.pallas import tpu_sc as plsc

N = 10000
D = 128
E = 320000
NC = 2
NS = 16
NW = NC * NS
CHUNK = 80
EPW = E // NW
CPW = EPW // CHUNK
RPT = N // NS
WB = 125
HW = 16
RB = 1000

_MESH = plsc.VectorSubcoreMesh(core_axis_name="c", subcore_axis_name="s",
                               num_cores=NC, num_subcores=NS)


@functools.partial(
    pl.kernel,
    out_type=jax.ShapeDtypeStruct((NC, N, HW), jnp.float32),
    mesh=_MESH,
    scratch_types=[
        pltpu.VMEM((CPW, CHUNK), jnp.int32),
        pltpu.VMEM((CHUNK, HW), jnp.float32),
        pltpu.VMEM((WB, HW), jnp.float32),
        pltpu.VMEM_SHARED((N, HW), jnp.float32),
    ],
)
def _hist(dst2_hbm, hout_hbm, idxv, onesv, zv, hacc):
    cid = lax.axis_index("c")
    sid = lax.axis_index("s")
    wid = cid * NS + sid
    ones16 = jnp.ones((16,), jnp.float32)
    zero16 = jnp.zeros((16,), jnp.float32)

    def fill_ones(i, c):
        onesv[i, :] = ones16
        return c
    lax.fori_loop(0, CHUNK, fill_ones, 0)

    def fill_zero(i, c):
        zv[i, :] = zero16
        return c
    lax.fori_loop(0, WB, fill_zero, 0)

    for j in range(RPT // WB):
        pltpu.sync_copy(zv, hacc.at[pl.ds(sid * RPT + j * WB, WB)])
    plsc.subcore_barrier()

    pltpu.sync_copy(dst2_hbm.at[pl.ds(wid * CPW, CPW)], idxv)

    def step(i, c):
        pltpu.sync_copy(onesv, hacc.at[idxv.at[i]], add=True)
        return c
    lax.fori_loop(0, CPW, step, 0)
    plsc.subcore_barrier()

    for j in range(RPT // WB):
        sl = pl.ds(sid * RPT + j * WB, WB)
        pltpu.sync_copy(hacc.at[sl], hout_hbm.at[cid, sl])


@functools.partial(
    pl.kernel,
    out_type=jax.ShapeDtypeStruct((NC, N, D), jnp.float32),
    mesh=_MESH,
    scratch_types=[
        pltpu.VMEM((CPW, CHUNK), jnp.int32),
        pltpu.VMEM((CPW, CHUNK), jnp.int32),
        pltpu.VMEM((CHUNK, D), jnp.float32),
        pltpu.VMEM((WB, D), jnp.float32),
        pltpu.VMEM_SHARED((N, D), jnp.float32),
        pltpu.SemaphoreType.DMA,
    ],
)
def _scat(xs_hbm, src2_hbm, dst2_hbm, part_hbm, srcv, dstv, rows, zv, acc, sem):
    cid = lax.axis_index("c")
    sid = lax.axis_index("s")
    wid = cid * NS + sid
    zero16 = jnp.zeros((16,), jnp.float32)

    def fill_zero(i, c):
        for k in range(D // 16):
            zv[i, pl.ds(k * 16, 16)] = zero16
        return c
    lax.fori_loop(0, WB, fill_zero, 0)

    for j in range(RPT // WB):
        pltpu.sync_copy(zv, acc.at[pl.ds(sid * RPT + j * WB, WB)])
    plsc.subcore_barrier()

    pltpu.sync_copy(src2_hbm.at[pl.ds(wid * CPW, CPW)], srcv)
    pltpu.sync_copy(dst2_hbm.at[pl.ds(wid * CPW, CPW)], dstv)

    def step(i, c):
        pltpu.async_copy(xs_hbm.at[srcv.at[i]], rows, sem).wait()
        pltpu.sync_copy(rows, acc.at[dstv.at[i]], add=True)
        return c
    lax.fori_loop(0, CPW, step, 0)
    plsc.subcore_barrier()

    for j in range(RPT // WB):
        sl = pl.ds(sid * RPT + j * WB, WB)
        pltpu.sync_copy(acc.at[sl], part_hbm.at[cid, sl])


def _prep_body(h0_ref, h1_ref, x_ref, w_ref, xs_ref):
    deg = h0_ref[:, 0:1] + h1_ref[:, 0:1] + 1.0
    dinv = lax.rsqrt(deg)
    xh = jnp.dot(x_ref[...], w_ref[...], preferred_element_type=jnp.float32)
    xs_ref[...] = xh * dinv


def _final_body(h0_ref, h1_ref, p0_ref, p1_ref, xs_ref, out_ref):
    deg = h0_ref[:, 0:1] + h1_ref[:, 0:1] + 1.0
    dinv = lax.rsqrt(deg)
    out_ref[...] = (p0_ref[...] + p1_ref[...] + xs_ref[...]) * dinv


def kernel(node_features, edge_list, W):
    src2 = edge_list[0].reshape(E // CHUNK, CHUNK)
    dst2 = edge_list[1].reshape(E // CHUNK, CHUNK)

    hparts = _hist(dst2)
    h0, h1 = hparts[0], hparts[1]

    hblk = pl.BlockSpec((RB, HW), lambda i: (i, 0))
    xs = pl.pallas_call(
        _prep_body,
        grid=(N // RB,),
        in_specs=[
            hblk,
            hblk,
            pl.BlockSpec((RB, D), lambda i: (i, 0)),
            pl.BlockSpec((D, D), lambda i: (0, 0)),
        ],
        out_specs=pl.BlockSpec((RB, D), lambda i: (i, 0)),
        out_shape=jax.ShapeDtypeStruct((N, D), jnp.float32),
    )(h0, h1, node_features, W)

    parts = _scat(xs, src2, dst2)

    fblk = pl.BlockSpec((RB, D), lambda i: (i, 0))
    out = pl.pallas_call(
        _final_body,
        grid=(N // RB,),
        in_specs=[hblk, hblk, fblk, fblk, fblk],
        out_specs=fblk,
        out_shape=jax.ShapeDtypeStruct((N, D), jnp.float32),
    )(h0, h1, parts[0], parts[1], xs)
    return out


# speedup vs baseline: 24.4264x; 24.4264x over previous
import functools

import jax
import jax.numpy as jnp
from jax import lax
from jax.experimental import pallas as pl
from jax.experimental.pallas import tpu as pltpu
from jax.experimental.pallas import tpu_sc as plsc

N = 10000
D = 128
E = 320000
NC = 2
NS = 16
NW = NC * NS
CHUNK = 125
EPW = E // NW
CPW = EPW // CHUNK
STG = 16
NSTG = CPW // STG
NPAD = 10240
RPT = NPAD // NS
WB = 128
HW = 16
RB = 1000

_MESH = plsc.VectorSubcoreMesh(core_axis_name="c", subcore_axis_name="s",
                               num_cores=NC, num_subcores=NS)


_HIST_SCRATCH = [
    pltpu.VMEM((STG, CHUNK), jnp.int32),
    pltpu.VMEM((WB, D), jnp.float32),
    pltpu.VMEM_SHARED((NPAD, D), jnp.float32),
]


def _hist_body(dst2_hbm, hout_hbm, dstv, rows, hacc):
    cid = lax.axis_index("c")
    sid = lax.axis_index("s")
    wid = cid * NS + sid
    zero16 = jnp.zeros((16,), jnp.float32)
    ones16 = jnp.ones((16,), jnp.float32)

    def fill_zero(i, c):
        for k in range(D // 16):
            rows[i, pl.ds(k * 16, 16)] = zero16
        return c
    lax.fori_loop(0, WB, fill_zero, 0)

    for j in range(RPT // WB):
        pltpu.sync_copy(rows, hacc.at[pl.ds(sid * RPT + j * WB, WB)])
    plsc.subcore_barrier()

    def fill_ones(i, c):
        for k in range(D // 16):
            rows[i, pl.ds(k * 16, 16)] = ones16
        return c
    lax.fori_loop(0, CHUNK, fill_ones, 0)

    def stage(j, c):
        pltpu.sync_copy(dst2_hbm.at[wid, pl.ds(j * STG, STG)], dstv)

        def step(i, c2):
            pltpu.sync_copy(rows.at[pl.ds(0, CHUNK)], hacc.at[dstv.at[i]], add=True)
            return c2
        lax.fori_loop(0, STG, step, 0)
        return c
    lax.fori_loop(0, NSTG, stage, 0)
    plsc.subcore_barrier()

    for j in range(RPT // WB):
        sl = pl.ds(sid * RPT + j * WB, WB)
        pltpu.sync_copy(hacc.at[sl], hout_hbm.at[cid, sl])


_hist = pl.kernel(
    _hist_body,
    out_type=jax.ShapeDtypeStruct((NC, NPAD, D), jnp.float32),
    mesh=_MESH,
    scratch_types=_HIST_SCRATCH,
)


_SCAT_SCRATCH = [
    pltpu.VMEM((STG, CHUNK), jnp.int32),
    pltpu.VMEM((STG, CHUNK), jnp.int32),
    pltpu.VMEM((WB, D), jnp.float32),
    pltpu.VMEM_SHARED((NPAD, D), jnp.float32),
    pltpu.SemaphoreType.DMA,
]


def _scat_body(xs_hbm, src2_hbm, dst2_hbm, part_hbm, srcv, dstv, rows, acc, sem):
    cid = lax.axis_index("c")
    sid = lax.axis_index("s")
    wid = cid * NS + sid
    zero16 = jnp.zeros((16,), jnp.float32)

    def fill_zero(i, c):
        for k in range(D // 16):
            rows[i, pl.ds(k * 16, 16)] = zero16
        return c
    lax.fori_loop(0, WB, fill_zero, 0)

    for j in range(RPT // WB):
        pltpu.sync_copy(rows, acc.at[pl.ds(sid * RPT + j * WB, WB)])
    plsc.subcore_barrier()

    def stage(j, c):
        pltpu.sync_copy(src2_hbm.at[wid, pl.ds(j * STG, STG)], srcv)
        pltpu.sync_copy(dst2_hbm.at[wid, pl.ds(j * STG, STG)], dstv)

        def step(i, c2):
            pltpu.async_copy(xs_hbm.at[srcv.at[i]], rows.at[pl.ds(0, CHUNK)], sem).wait()
            pltpu.sync_copy(rows.at[pl.ds(0, CHUNK)], acc.at[dstv.at[i]], add=True)
            return c2
        lax.fori_loop(0, STG, step, 0)
        return c
    lax.fori_loop(0, NSTG, stage, 0)
    plsc.subcore_barrier()

    for j in range(RPT // WB):
        sl = pl.ds(sid * RPT + j * WB, WB)
        pltpu.sync_copy(acc.at[sl], part_hbm.at[cid, sl])


_scat = pl.kernel(
    _scat_body,
    out_type=jax.ShapeDtypeStruct((NC, NPAD, D), jnp.float32),
    mesh=_MESH,
    scratch_types=_SCAT_SCRATCH,
)


def _prep_body(h0_ref, h1_ref, x_ref, w_ref, xs_ref):
    deg = h0_ref[:, 0:1] + h1_ref[:, 0:1] + 1.0
    dinv = lax.rsqrt(deg)
    xh = jnp.dot(x_ref[...], w_ref[...], preferred_element_type=jnp.float32)
    xs_ref[...] = xh * dinv


def _final_body(h0_ref, h1_ref, p0_ref, p1_ref, xs_ref, out_ref):
    deg = h0_ref[:, 0:1] + h1_ref[:, 0:1] + 1.0
    dinv = lax.rsqrt(deg)
    out_ref[...] = (p0_ref[...] + p1_ref[...] + xs_ref[...]) * dinv


def kernel(node_features, edge_list, W):
    src2 = edge_list[0].reshape(NW, CPW, CHUNK)
    dst2 = edge_list[1].reshape(NW, CPW, CHUNK)

    hparts = _hist(dst2)
    h0, h1 = hparts[0], hparts[1]

    hblk = pl.BlockSpec((RB, D), lambda i: (i, 0))
    xs = pl.pallas_call(
        _prep_body,
        grid=(N // RB,),
        in_specs=[
            hblk,
            hblk,
            pl.BlockSpec((RB, D), lambda i: (i, 0)),
            pl.BlockSpec((D, D), lambda i: (0, 0)),
        ],
        out_specs=pl.BlockSpec((RB, D), lambda i: (i, 0)),
        out_shape=jax.ShapeDtypeStruct((N, D), jnp.float32),
    )(h0, h1, node_features, W)

    parts = _scat(xs, src2, dst2)

    fblk = pl.BlockSpec((RB, D), lambda i: (i, 0))
    out = pl.pallas_call(
        _final_body,
        grid=(N // RB,),
        in_specs=[hblk, hblk, fblk, fblk, fblk],
        out_specs=fblk,
        out_shape=jax.ShapeDtypeStruct((N, D), jnp.float32),
    )(h0, h1, parts[0], parts[1], xs)
    return out


# speedup vs baseline: 27.3722x; 1.1206x over previous
import functools

import jax
import jax.numpy as jnp
from jax import lax
from jax.experimental import pallas as pl
from jax.experimental.pallas import tpu as pltpu
from jax.experimental.pallas import tpu_sc as plsc

N = 10000
D = 128
E = 320000
NC = 2
NS = 16
NW = NC * NS
CHUNK = 125
EPW = E // NW
CPW = EPW // CHUNK
STG = 8
NSTG = CPW // STG
NPAD = 10240
RPT = NPAD // NS
WB = 128
HW = 16
RB = 1000

_MESH = plsc.VectorSubcoreMesh(core_axis_name="c", subcore_axis_name="s",
                               num_cores=NC, num_subcores=NS)


_HIST_SCRATCH = [
    pltpu.VMEM((STG, CHUNK), jnp.int32),
    pltpu.VMEM((WB, D), jnp.float32),
    pltpu.VMEM_SHARED((NPAD, D), jnp.float32),
]


def _hist_body(dst2_hbm, hout_hbm, dstv, rows, hacc):
    cid = lax.axis_index("c")
    sid = lax.axis_index("s")
    wid = cid * NS + sid
    zero16 = jnp.zeros((16,), jnp.float32)
    ones16 = jnp.ones((16,), jnp.float32)

    def fill_zero(i, c):
        for k in range(D // 16):
            rows[i, pl.ds(k * 16, 16)] = zero16
        return c
    lax.fori_loop(0, WB, fill_zero, 0)

    for j in range(RPT // WB):
        pltpu.sync_copy(rows, hacc.at[pl.ds(sid * RPT + j * WB, WB)])
    plsc.subcore_barrier()

    def fill_ones(i, c):
        for k in range(D // 16):
            rows[i, pl.ds(k * 16, 16)] = ones16
        return c
    lax.fori_loop(0, CHUNK, fill_ones, 0)

    def stage(j, c):
        pltpu.sync_copy(dst2_hbm.at[wid, pl.ds(j * STG, STG)], dstv)

        def step(i, c2):
            pltpu.sync_copy(rows.at[pl.ds(0, CHUNK)], hacc.at[dstv.at[i]], add=True)
            return c2
        lax.fori_loop(0, STG, step, 0)
        return c
    lax.fori_loop(0, NSTG, stage, 0)
    plsc.subcore_barrier()

    for j in range(RPT // WB):
        sl = pl.ds(sid * RPT + j * WB, WB)
        pltpu.sync_copy(hacc.at[sl], hout_hbm.at[cid, sl])


_hist = pl.kernel(
    _hist_body,
    out_type=jax.ShapeDtypeStruct((NC, NPAD, D), jnp.float32),
    mesh=_MESH,
    scratch_types=_HIST_SCRATCH,
)


_SCAT_SCRATCH = [
    pltpu.VMEM((STG, CHUNK), jnp.int32),
    pltpu.VMEM((STG, CHUNK), jnp.int32),
    pltpu.VMEM((WB, D), jnp.float32),
    pltpu.VMEM((CHUNK, D), jnp.float32),
    pltpu.VMEM_SHARED((NPAD, D), jnp.float32),
    pltpu.SemaphoreType.DMA,
    pltpu.SemaphoreType.DMA,
]


def _scat_body(xs_hbm, src2_hbm, dst2_hbm, part_hbm, srcv, dstv, rowsa, rowsb,
               acc, sema, semb):
    cid = lax.axis_index("c")
    sid = lax.axis_index("s")
    wid = cid * NS + sid
    zero16 = jnp.zeros((16,), jnp.float32)

    def fill_zero(i, c):
        for k in range(D // 16):
            rowsa[i, pl.ds(k * 16, 16)] = zero16
        return c
    lax.fori_loop(0, WB, fill_zero, 0)

    for j in range(RPT // WB):
        pltpu.sync_copy(rowsa, acc.at[pl.ds(sid * RPT + j * WB, WB)])
    plsc.subcore_barrier()

    bufs = (rowsa.at[pl.ds(0, CHUNK)], rowsb)
    sems = (sema, semb)

    def stage(j, c):
        pltpu.sync_copy(src2_hbm.at[wid, pl.ds(j * STG, STG)], srcv)
        pltpu.sync_copy(dst2_hbm.at[wid, pl.ds(j * STG, STG)], dstv)
        pend = pltpu.async_copy(xs_hbm.at[srcv.at[0]], bufs[0], sems[0])
        for k in range(STG):
            pend.wait()
            if k + 1 < STG:
                pend = pltpu.async_copy(xs_hbm.at[srcv.at[k + 1]],
                                        bufs[(k + 1) % 2], sems[(k + 1) % 2])
            pltpu.sync_copy(bufs[k % 2], acc.at[dstv.at[k]], add=True)
        return c
    lax.fori_loop(0, NSTG, stage, 0)
    plsc.subcore_barrier()

    for j in range(RPT // WB):
        sl = pl.ds(sid * RPT + j * WB, WB)
        pltpu.sync_copy(acc.at[sl], part_hbm.at[cid, sl])


_scat = pl.kernel(
    _scat_body,
    out_type=jax.ShapeDtypeStruct((NC, NPAD, D), jnp.float32),
    mesh=_MESH,
    scratch_types=_SCAT_SCRATCH,
)


def _prep_body(h0_ref, h1_ref, x_ref, w_ref, xs_ref):
    deg = h0_ref[:, 0:1] + h1_ref[:, 0:1] + 1.0
    dinv = lax.rsqrt(deg)
    xh = jnp.dot(x_ref[...], w_ref[...], preferred_element_type=jnp.float32)
    xs_ref[...] = xh * dinv


def _final_body(h0_ref, h1_ref, p0_ref, p1_ref, xs_ref, out_ref):
    deg = h0_ref[:, 0:1] + h1_ref[:, 0:1] + 1.0
    dinv = lax.rsqrt(deg)
    out_ref[...] = (p0_ref[...] + p1_ref[...] + xs_ref[...]) * dinv


def kernel(node_features, edge_list, W):
    src2 = edge_list[0].reshape(NW, CPW, CHUNK)
    dst2 = edge_list[1].reshape(NW, CPW, CHUNK)

    hparts = _hist(dst2)
    h0, h1 = hparts[0], hparts[1]

    hblk = pl.BlockSpec((RB, D), lambda i: (i, 0))
    xs = pl.pallas_call(
        _prep_body,
        grid=(N // RB,),
        in_specs=[
            hblk,
            hblk,
            pl.BlockSpec((RB, D), lambda i: (i, 0)),
            pl.BlockSpec((D, D), lambda i: (0, 0)),
        ],
        out_specs=pl.BlockSpec((RB, D), lambda i: (i, 0)),
        out_shape=jax.ShapeDtypeStruct((N, D), jnp.float32),
    )(h0, h1, node_features, W)

    parts = _scat(xs, src2, dst2)

    fblk = pl.BlockSpec((RB, D), lambda i: (i, 0))
    out = pl.pallas_call(
        _final_body,
        grid=(N // RB,),
        in_specs=[hblk, hblk, fblk, fblk, fblk],
        out_specs=fblk,
        out_shape=jax.ShapeDtypeStruct((N, D), jnp.float32),
    )(h0, h1, parts[0], parts[1], xs)
    return out
